# baseline (speedup 1.0000x reference)
import jax
import jax.numpy as jnp
from jax.experimental import pallas as pl
from jax.experimental.pallas import tpu as pltpu

_Q = 256
_D = 16
_GD = 128
_N = 100000
_K = 10
_CHUNK = 12800
_NCHUNKS = 8
_NPAD = _CHUNK * _NCHUNKS
_NSLICE = _CHUNK // 128
_INF = float("inf")


def _q2(q_ref):
    q = q_ref[...]
    qn = q / (jnp.sqrt(jnp.sum(q * q, axis=1, keepdims=True)) + 1e-8)
    return jnp.sum(qn * qn, axis=1, keepdims=True)


def _partial_distances(q_ref, s_ref, c):
    q = q_ref[...]
    qn = q / (jnp.sqrt(jnp.sum(q * q, axis=1, keepdims=True)) + 1e-8)
    s = s_ref[...]
    sn = s / (jnp.sqrt(jnp.sum(s * s, axis=0, keepdims=True)) + 1e-8)
    s2 = jnp.sum(sn * sn, axis=0, keepdims=True)
    lane = jax.lax.broadcasted_iota(jnp.int32, (1, _CHUNK), 1)
    s2 = jnp.where(lane < (_N - c * _CHUNK), s2, _INF)
    dots = jnp.dot(qn, sn, preferred_element_type=jnp.float32)
    return s2 - 2.0 * dots


def _fold_top3_max(dots):
    m1 = jnp.full((_Q, 128), -_INF, jnp.float32)
    m2 = jnp.full((_Q, 128), -_INF, jnp.float32)
    m3 = jnp.full((_Q, 128), -_INF, jnp.float32)
    for j in range(_NSLICE):
        v = dots[:, j * 128:(j + 1) * 128]
        t1 = jnp.minimum(m1, v)
        m1 = jnp.maximum(m1, v)
        t2 = jnp.minimum(m2, t1)
        m2 = jnp.maximum(m2, t1)
        m3 = jnp.maximum(m3, t2)
    return m1, m2, m3


def _main_body(q_ref, s_ref, ge_ref, gmin_ref, gmax_ref, thr_ref,
               avg_ref, mask_ref, frac_ref, flag_ref,
               cv1_ref, cv2_ref, cv3_ref, dev_ref):
    i = pl.program_id(0)

    @pl.when(i == 0)
    def _init():
        ge = ge_ref[...]
        oob = ((ge < gmin_ref[...]) | (ge > gmax_ref[...])).astype(jnp.float32)
        frac = jnp.sum(oob) * (1.0 / (_Q * _GD))
        frac_ref[...] = jnp.zeros((1, 128), jnp.float32) + frac
        dev_ref[...] = jnp.zeros((1, 128), jnp.float32)

    q = q_ref[...]
    qn = q / (jnp.sqrt(jnp.sum(q * q, axis=1, keepdims=True)) + 1e-8)
    s = s_ref[...]
    s2_raw = jnp.sum(s * s, axis=0, keepdims=True)
    inv = 1.0 / (jnp.sqrt(s2_raw) + 1e-8)
    sn = s * inv
    s2 = s2_raw * inv * inv
    lane = jax.lax.broadcasted_iota(jnp.int32, (1, _CHUNK), 1)
    valid = lane < (_N - i * _CHUNK)
    dev = jnp.max(jnp.where(valid, jnp.abs(s2 - 1.0), 0.0))
    dev_ref[...] = jnp.maximum(dev_ref[...], jnp.zeros((1, 128)) + dev)
    dots = jnp.dot(qn, sn, preferred_element_type=jnp.float32)

    @pl.when(i < _NCHUNKS - 1)
    def _fold_plain():
        m1, m2, m3 = _fold_top3_max(dots)
        cv1_ref[i] = m1
        cv2_ref[i] = m2
        cv3_ref[i] = m3

    @pl.when(i == _NCHUNKS - 1)
    def _fold_masked():
        m1, m2, m3 = _fold_top3_max(jnp.where(valid, dots, -_INF))
        cv1_ref[i] = m1
        cv2_ref[i] = m2
        cv3_ref[i] = m3

    @pl.when(i == _NCHUNKS - 1)
    def _merge():
        a = jnp.concatenate([cv1_ref[j] for j in range(_NCHUNKS)], axis=1)
        b = jnp.concatenate([cv2_ref[j] for j in range(_NCHUNKS)], axis=1)
        c3 = jnp.concatenate([cv3_ref[j] for j in range(_NCHUNKS)], axis=1)
        m3cat = c3
        q2 = _q2(q_ref)
        total = jnp.zeros((_Q, 1), jnp.float32)
        acc = jnp.zeros((_Q, 1), jnp.float32)
        cum = jnp.zeros((_Q, 1), jnp.float32)
        tau = jnp.full((_Q, 1), -_INF, jnp.float32)
        for _ in range(_K):
            m = jnp.max(a, axis=1, keepdims=True)
            eq = a == m
            cnt = jnp.sum(eq.astype(jnp.float32), axis=1, keepdims=True)
            a = jnp.where(eq, b, a)
            b = jnp.where(eq, c3, b)
            c3 = jnp.where(eq, -_INF, c3)
            cum = cum + cnt
            tau = jnp.maximum(tau, jnp.where(cum >= _K, m, -_INF))
            take = jnp.clip(jnp.minimum(cnt, _K - total), 0.0, None)
            dm = jnp.where(
                m > -_INF,
                jnp.sqrt(jnp.maximum(q2 + 1.0 - 2.0 * m, 0.0) + 1e-12), 0.0)
            acc = acc + take * dm
            total = total + take
        avg = acc * (1.0 / _K)
        avg_ref[...] = jnp.broadcast_to(avg, (_Q, 128))
        mask = (avg > thr_ref[0, 0]).astype(jnp.float32)
        mask_ref[...] = jnp.broadcast_to(mask, (_Q, 128))
        bad = jnp.sum((m3cat >= tau).astype(jnp.float32))
        bad = bad + jnp.sum((dev_ref[...] > 1e-5).astype(jnp.float32))
        flag_ref[...] = jnp.zeros((1, 128), jnp.float32) + (bad > 0.0)


def _fallback_body(q_ref, s_ref, thr_ref, avg_ref, mask_ref, run_ref):
    i = pl.program_id(0)

    @pl.when(i == 0)
    def _init():
        run_ref[...] = jnp.full((_Q, 16), _INF, jnp.float32)

    rem = _partial_distances(q_ref, s_ref, i)
    run = run_ref[...]
    ms, ccs = [], []
    cum = jnp.zeros((_Q, 1), jnp.float32)
    for _ in range(_K):
        m = jnp.minimum(jnp.min(rem, axis=1, keepdims=True),
                        jnp.min(run, axis=1, keepdims=True))
        eqc = rem == m
        eqr = run == m
        cnt = (jnp.sum(eqc.astype(jnp.float32), axis=1, keepdims=True)
               + jnp.sum(eqr.astype(jnp.float32), axis=1, keepdims=True))
        rem = jnp.where(eqc, _INF, rem)
        run = jnp.where(eqr, _INF, run)
        cum = cum + cnt
        ms.append(m)
        ccs.append(cum)
    mvals = jnp.concatenate(ms, axis=1)
    ccum = jnp.concatenate(ccs, axis=1)
    cols = [jnp.min(jnp.where(ccum > j, mvals, _INF), axis=1, keepdims=True)
            for j in range(_K)]
    cols += [jnp.full((_Q, 1), _INF, jnp.float32)] * (16 - _K)
    new_run = jnp.concatenate(cols, axis=1)
    run_ref[...] = new_run

    @pl.when(i == _NCHUNKS - 1)
    def _fini():
        d = jnp.sqrt(jnp.maximum(new_run[:, :_K] + _q2(q_ref), 0.0) + 1e-12)
        avg = jnp.sum(d, axis=1, keepdims=True) * (1.0 / _K)
        avg_ref[...] = jnp.broadcast_to(avg, (_Q, 128))
        mask = (avg > thr_ref[0, 0]).astype(jnp.float32)
        mask_ref[...] = jnp.broadcast_to(mask, (_Q, 128))


def kernel(global_embedding, geometry_latent, global_min, global_max,
           geo_embeddings, knn_threshold):
    geo_t = jnp.pad(geo_embeddings, ((0, _NPAD - _N), (0, 0))).T
    gmin = global_min.reshape(1, _GD)
    gmax = global_max.reshape(1, _GD)
    thr = jnp.asarray(knn_threshold, jnp.float32).reshape(1, 1)

    avg_b, mask_b, frac_b, flag_b = pl.pallas_call(
        _main_body,
        grid=(_NCHUNKS,),
        in_specs=[
            pl.BlockSpec((_Q, _D), lambda i: (0, 0)),
            pl.BlockSpec((_D, _CHUNK), lambda i: (0, i)),
            pl.BlockSpec((_Q, _GD), lambda i: (0, 0)),
            pl.BlockSpec((1, _GD), lambda i: (0, 0)),
            pl.BlockSpec((1, _GD), lambda i: (0, 0)),
            pl.BlockSpec((1, 1), lambda i: (0, 0)),
        ],
        out_specs=[
            pl.BlockSpec((_Q, 128), lambda i: (0, 0)),
            pl.BlockSpec((_Q, 128), lambda i: (0, 0)),
            pl.BlockSpec((1, 128), lambda i: (0, 0)),
            pl.BlockSpec((1, 128), lambda i: (0, 0)),
        ],
        out_shape=[
            jax.ShapeDtypeStruct((_Q, 128), jnp.float32),
            jax.ShapeDtypeStruct((_Q, 128), jnp.float32),
            jax.ShapeDtypeStruct((1, 128), jnp.float32),
            jax.ShapeDtypeStruct((1, 128), jnp.float32),
        ],
        scratch_shapes=[
            pltpu.VMEM((_NCHUNKS, _Q, 128), jnp.float32),
            pltpu.VMEM((_NCHUNKS, _Q, 128), jnp.float32),
            pltpu.VMEM((_NCHUNKS, _Q, 128), jnp.float32),
            pltpu.VMEM((1, 128), jnp.float32),
        ],
    )(geometry_latent, geo_t, global_embedding, gmin, gmax, thr)

    def _run_fallback(_):
        return tuple(pl.pallas_call(
            _fallback_body,
            grid=(_NCHUNKS,),
            in_specs=[
                pl.BlockSpec((_Q, _D), lambda i: (0, 0)),
                pl.BlockSpec((_D, _CHUNK), lambda i: (0, i)),
                pl.BlockSpec((1, 1), lambda i: (0, 0)),
            ],
            out_specs=[
                pl.BlockSpec((_Q, 128), lambda i: (0, 0)),
                pl.BlockSpec((_Q, 128), lambda i: (0, 0)),
            ],
            out_shape=[
                jax.ShapeDtypeStruct((_Q, 128), jnp.float32),
                jax.ShapeDtypeStruct((_Q, 128), jnp.float32),
            ],
            scratch_shapes=[pltpu.VMEM((_Q, 16), jnp.float32)],
        )(geometry_latent, geo_t, thr))

    avg_b, mask_b = jax.lax.cond(
        flag_b[0, 0] > 0.5, _run_fallback, lambda _: (avg_b, mask_b), None)

    avg = avg_b[:, 0]
    ood_mask = mask_b[:, 0].astype(bool)
    frac_oob = frac_b[0, 0]
    return (avg, ood_mask, frac_oob)


# speedup vs baseline: 1.0101x; 1.0101x over previous
import jax
import jax.numpy as jnp
from jax.experimental import pallas as pl
from jax.experimental.pallas import tpu as pltpu

_Q = 256
_D = 16
_GD = 128
_N = 100000
_K = 10
_CHUNK = 12800
_NCHUNKS = 8
_NPAD = _CHUNK * _NCHUNKS
_NSLICE = _CHUNK // 128
_INF = float("inf")


def _q2(q_ref):
    q = q_ref[...]
    qn = q / (jnp.sqrt(jnp.sum(q * q, axis=1, keepdims=True)) + 1e-8)
    return jnp.sum(qn * qn, axis=1, keepdims=True)


def _partial_distances(q_ref, s_ref, c):
    q = q_ref[...]
    qn = q / (jnp.sqrt(jnp.sum(q * q, axis=1, keepdims=True)) + 1e-8)
    s = s_ref[...]
    sn = s / (jnp.sqrt(jnp.sum(s * s, axis=0, keepdims=True)) + 1e-8)
    s2 = jnp.sum(sn * sn, axis=0, keepdims=True)
    lane = jax.lax.broadcasted_iota(jnp.int32, (1, _CHUNK), 1)
    s2 = jnp.where(lane < (_N - c * _CHUNK), s2, _INF)
    dots = jnp.dot(qn, sn, preferred_element_type=jnp.float32)
    return s2 - 2.0 * dots


def _fold_top3_max(dots):
    m1 = jnp.full((_Q, 128), -_INF, jnp.float32)
    m2 = jnp.full((_Q, 128), -_INF, jnp.float32)
    m3 = jnp.full((_Q, 128), -_INF, jnp.float32)
    for j in range(_NSLICE):
        v = dots[:, j * 128:(j + 1) * 128]
        t1 = jnp.minimum(m1, v)
        m1 = jnp.maximum(m1, v)
        t2 = jnp.minimum(m2, t1)
        m2 = jnp.maximum(m2, t1)
        m3 = jnp.maximum(m3, t2)
    return m1, m2, m3


def _main_body(q_ref, s_ref, ge_ref, gmin_ref, gmax_ref, thr_ref,
               avg_ref, mask_ref, frac_ref, flag_ref,
               cv1_ref, cv2_ref, cv3_ref, dev_ref):
    i = pl.program_id(0)

    @pl.when(i == 0)
    def _init():
        ge = ge_ref[...]
        oob = ((ge < gmin_ref[...]) | (ge > gmax_ref[...])).astype(jnp.float32)
        frac = jnp.sum(oob) * (1.0 / (_Q * _GD))
        frac_ref[...] = jnp.zeros((1, 128), jnp.float32) + frac
        dev_ref[...] = jnp.zeros((1, 128), jnp.float32)

    q = q_ref[...]
    qn = q / (jnp.sqrt(jnp.sum(q * q, axis=1, keepdims=True)) + 1e-8)
    s = s_ref[...]
    s2_raw = jnp.sum(s * s, axis=0, keepdims=True)
    inv = 1.0 / (jnp.sqrt(s2_raw) + 1e-8)
    sn = s * inv
    s2 = s2_raw * inv * inv
    lane = jax.lax.broadcasted_iota(jnp.int32, (1, _CHUNK), 1)
    valid = lane < (_N - i * _CHUNK)
    deva = jnp.where(valid, jnp.abs(s2 - 1.0), 0.0)
    devrow = deva[:, :128]
    for j in range(1, _NSLICE):
        devrow = jnp.maximum(devrow, deva[:, j * 128:(j + 1) * 128])
    dev_ref[...] = jnp.maximum(dev_ref[...], devrow)
    dots = jnp.dot(qn, sn, preferred_element_type=jnp.float32)

    @pl.when(i < _NCHUNKS - 1)
    def _fold_plain():
        m1, m2, m3 = _fold_top3_max(dots)
        cv1_ref[i] = m1
        cv2_ref[i] = m2
        cv3_ref[i] = m3

    @pl.when(i == _NCHUNKS - 1)
    def _fold_masked():
        m1, m2, m3 = _fold_top3_max(jnp.where(valid, dots, -_INF))
        cv1_ref[i] = m1
        cv2_ref[i] = m2
        cv3_ref[i] = m3

    @pl.when(i == _NCHUNKS - 1)
    def _merge():
        a = jnp.concatenate([cv1_ref[j] for j in range(_NCHUNKS)], axis=1)
        b = jnp.concatenate([cv2_ref[j] for j in range(_NCHUNKS)], axis=1)
        c3 = jnp.concatenate([cv3_ref[j] for j in range(_NCHUNKS)], axis=1)
        m3cat = c3
        q2 = _q2(q_ref)
        total = jnp.zeros((_Q, 1), jnp.float32)
        acc = jnp.zeros((_Q, 1), jnp.float32)
        cum = jnp.zeros((_Q, 1), jnp.float32)
        tau = jnp.full((_Q, 1), -_INF, jnp.float32)
        for _ in range(_K):
            m = jnp.max(a, axis=1, keepdims=True)
            eq = a == m
            cnt = jnp.sum(eq.astype(jnp.float32), axis=1, keepdims=True)
            a = jnp.where(eq, b, a)
            b = jnp.where(eq, c3, b)
            c3 = jnp.where(eq, -_INF, c3)
            cum = cum + cnt
            tau = jnp.maximum(tau, jnp.where(cum >= _K, m, -_INF))
            take = jnp.clip(jnp.minimum(cnt, _K - total), 0.0, None)
            dm = jnp.where(
                m > -_INF,
                jnp.sqrt(jnp.maximum(q2 + 1.0 - 2.0 * m, 0.0) + 1e-12), 0.0)
            acc = acc + take * dm
            total = total + take
        avg = acc * (1.0 / _K)
        avg_ref[...] = jnp.broadcast_to(avg, (_Q, 128))
        mask = (avg > thr_ref[0, 0]).astype(jnp.float32)
        mask_ref[...] = jnp.broadcast_to(mask, (_Q, 128))
        bad = jnp.sum((m3cat >= tau).astype(jnp.float32))
        bad = bad + jnp.sum((dev_ref[...] > 1e-5).astype(jnp.float32))
        flag_ref[...] = jnp.zeros((1, 128), jnp.float32) + (bad > 0.0)


def _fallback_body(q_ref, s_ref, thr_ref, avg_ref, mask_ref, run_ref):
    i = pl.program_id(0)

    @pl.when(i == 0)
    def _init():
        run_ref[...] = jnp.full((_Q, 16), _INF, jnp.float32)

    rem = _partial_distances(q_ref, s_ref, i)
    run = run_ref[...]
    ms, ccs = [], []
    cum = jnp.zeros((_Q, 1), jnp.float32)
    for _ in range(_K):
        m = jnp.minimum(jnp.min(rem, axis=1, keepdims=True),
                        jnp.min(run, axis=1, keepdims=True))
        eqc = rem == m
        eqr = run == m
        cnt = (jnp.sum(eqc.astype(jnp.float32), axis=1, keepdims=True)
               + jnp.sum(eqr.astype(jnp.float32), axis=1, keepdims=True))
        rem = jnp.where(eqc, _INF, rem)
        run = jnp.where(eqr, _INF, run)
        cum = cum + cnt
        ms.append(m)
        ccs.append(cum)
    mvals = jnp.concatenate(ms, axis=1)
    ccum = jnp.concatenate(ccs, axis=1)
    cols = [jnp.min(jnp.where(ccum > j, mvals, _INF), axis=1, keepdims=True)
            for j in range(_K)]
    cols += [jnp.full((_Q, 1), _INF, jnp.float32)] * (16 - _K)
    new_run = jnp.concatenate(cols, axis=1)
    run_ref[...] = new_run

    @pl.when(i == _NCHUNKS - 1)
    def _fini():
        d = jnp.sqrt(jnp.maximum(new_run[:, :_K] + _q2(q_ref), 0.0) + 1e-12)
        avg = jnp.sum(d, axis=1, keepdims=True) * (1.0 / _K)
        avg_ref[...] = jnp.broadcast_to(avg, (_Q, 128))
        mask = (avg > thr_ref[0, 0]).astype(jnp.float32)
        mask_ref[...] = jnp.broadcast_to(mask, (_Q, 128))


def kernel(global_embedding, geometry_latent, global_min, global_max,
           geo_embeddings, knn_threshold):
    geo_t = jnp.pad(geo_embeddings, ((0, _NPAD - _N), (0, 0))).T
    gmin = global_min.reshape(1, _GD)
    gmax = global_max.reshape(1, _GD)
    thr = jnp.asarray(knn_threshold, jnp.float32).reshape(1, 1)

    avg_b, mask_b, frac_b, flag_b = pl.pallas_call(
        _main_body,
        grid=(_NCHUNKS,),
        in_specs=[
            pl.BlockSpec((_Q, _D), lambda i: (0, 0)),
            pl.BlockSpec((_D, _CHUNK), lambda i: (0, i)),
            pl.BlockSpec((_Q, _GD), lambda i: (0, 0)),
            pl.BlockSpec((1, _GD), lambda i: (0, 0)),
            pl.BlockSpec((1, _GD), lambda i: (0, 0)),
            pl.BlockSpec((1, 1), lambda i: (0, 0)),
        ],
        out_specs=[
            pl.BlockSpec((_Q, 128), lambda i: (0, 0)),
            pl.BlockSpec((_Q, 128), lambda i: (0, 0)),
            pl.BlockSpec((1, 128), lambda i: (0, 0)),
            pl.BlockSpec((1, 128), lambda i: (0, 0)),
        ],
        out_shape=[
            jax.ShapeDtypeStruct((_Q, 128), jnp.float32),
            jax.ShapeDtypeStruct((_Q, 128), jnp.float32),
            jax.ShapeDtypeStruct((1, 128), jnp.float32),
            jax.ShapeDtypeStruct((1, 128), jnp.float32),
        ],
        scratch_shapes=[
            pltpu.VMEM((_NCHUNKS, _Q, 128), jnp.float32),
            pltpu.VMEM((_NCHUNKS, _Q, 128), jnp.float32),
            pltpu.VMEM((_NCHUNKS, _Q, 128), jnp.float32),
            pltpu.VMEM((1, 128), jnp.float32),
        ],
    )(geometry_latent, geo_t, global_embedding, gmin, gmax, thr)

    def _run_fallback(_):
        return tuple(pl.pallas_call(
            _fallback_body,
            grid=(_NCHUNKS,),
            in_specs=[
                pl.BlockSpec((_Q, _D), lambda i: (0, 0)),
                pl.BlockSpec((_D, _CHUNK), lambda i: (0, i)),
                pl.BlockSpec((1, 1), lambda i: (0, 0)),
            ],
            out_specs=[
                pl.BlockSpec((_Q, 128), lambda i: (0, 0)),
                pl.BlockSpec((_Q, 128), lambda i: (0, 0)),
            ],
            out_shape=[
                jax.ShapeDtypeStruct((_Q, 128), jnp.float32),
                jax.ShapeDtypeStruct((_Q, 128), jnp.float32),
            ],
            scratch_shapes=[pltpu.VMEM((_Q, 16), jnp.float32)],
        )(geometry_latent, geo_t, thr))

    avg_b, mask_b = jax.lax.cond(
        flag_b[0, 0] > 0.5, _run_fallback, lambda _: (avg_b, mask_b), None)

    avg = avg_b[:, 0]
    ood_mask = mask_b[:, 0].astype(bool)
    frac_oob = frac_b[0, 0]
    return (avg, ood_mask, frac_oob)


# speedup vs baseline: 1.2887x; 1.2759x over previous
import jax
import jax.numpy as jnp
from jax.experimental import pallas as pl
from jax.experimental.pallas import tpu as pltpu

_Q = 256
_D = 16
_GD = 128
_N = 100000
_K = 10
_CHUNK = 12800
_NCHUNKS = 8
_NPAD = _CHUNK * _NCHUNKS
_NSLICE = _CHUNK // 128
_INF = float("inf")


def _q2(q_ref):
    q = q_ref[...]
    qn = q / (jnp.sqrt(jnp.sum(q * q, axis=1, keepdims=True)) + 1e-8)
    return jnp.sum(qn * qn, axis=1, keepdims=True)


def _partial_distances(q_ref, s_ref, c):
    q = q_ref[...]
    qn = q / (jnp.sqrt(jnp.sum(q * q, axis=1, keepdims=True)) + 1e-8)
    s = s_ref[...]
    s2_raw = jnp.sum(s * s, axis=0, keepdims=True)
    inv = 1.0 / (jnp.sqrt(s2_raw) + 1e-8)
    sn = s * inv
    s2 = s2_raw * inv * inv
    lane = jax.lax.broadcasted_iota(jnp.int32, (1, _CHUNK), 1)
    s2 = jnp.where(lane < (_N - c * _CHUNK), s2, _INF)
    dots = jnp.dot(qn, sn, preferred_element_type=jnp.float32)
    return s2 - 2.0 * dots


def _main_body(q_ref, s_ref, ge_ref, gmin_ref, gmax_ref, thr_ref,
               avg_ref, mask_ref, frac_ref, flag_ref,
               cv1_ref, cv2_ref, cv3_ref):
    i = pl.program_id(0)

    @pl.when(i == 0)
    def _init():
        ge = ge_ref[...]
        oob = ((ge < gmin_ref[...]) | (ge > gmax_ref[...])).astype(jnp.float32)
        frac = jnp.sum(oob) * (1.0 / (_Q * _GD))
        frac_ref[...] = jnp.zeros((1, 128), jnp.float32) + frac

    e = _partial_distances(q_ref, s_ref, i)
    m1 = jnp.full((_Q, 128), _INF, jnp.float32)
    m2 = jnp.full((_Q, 128), _INF, jnp.float32)
    m3 = jnp.full((_Q, 128), _INF, jnp.float32)
    for j in range(_NSLICE):
        v = e[:, j * 128:(j + 1) * 128]
        t1 = jnp.maximum(m1, v)
        m1 = jnp.minimum(m1, v)
        t2 = jnp.maximum(m2, t1)
        m2 = jnp.minimum(m2, t1)
        m3 = jnp.minimum(m3, t2)
    cv1_ref[i] = m1
    cv2_ref[i] = m2
    cv3_ref[i] = m3

    @pl.when(i == _NCHUNKS - 1)
    def _merge():
        a = jnp.concatenate([cv1_ref[j] for j in range(_NCHUNKS)], axis=1)
        b = jnp.concatenate([cv2_ref[j] for j in range(_NCHUNKS)], axis=1)
        c3 = jnp.concatenate([cv3_ref[j] for j in range(_NCHUNKS)], axis=1)
        m3cat = c3
        q2 = _q2(q_ref)
        total = jnp.zeros((_Q, 1), jnp.float32)
        acc = jnp.zeros((_Q, 1), jnp.float32)
        cum = jnp.zeros((_Q, 1), jnp.float32)
        tau = jnp.full((_Q, 1), _INF, jnp.float32)
        for _ in range(_K):
            m = jnp.min(a, axis=1, keepdims=True)
            eq = a == m
            cnt = jnp.sum(eq.astype(jnp.float32), axis=1, keepdims=True)
            a = jnp.where(eq, b, a)
            b = jnp.where(eq, c3, b)
            c3 = jnp.where(eq, _INF, c3)
            cum = cum + cnt
            tau = jnp.minimum(tau, jnp.where(cum >= _K, m, _INF))
            take = jnp.clip(jnp.minimum(cnt, _K - total), 0.0, None)
            dm = jnp.where(m < _INF,
                           jnp.sqrt(jnp.maximum(m + q2, 0.0) + 1e-12), 0.0)
            acc = acc + take * dm
            total = total + take
        avg = acc * (1.0 / _K)
        avg_ref[...] = jnp.broadcast_to(avg, (_Q, 128))
        mask = (avg > thr_ref[0, 0]).astype(jnp.float32)
        mask_ref[...] = jnp.broadcast_to(mask, (_Q, 128))
        bad = jnp.sum((m3cat <= tau).astype(jnp.float32))
        flag_ref[...] = jnp.zeros((1, 128), jnp.float32) + (bad > 0.0)


def _fallback_body(q_ref, s_ref, thr_ref, avg_ref, mask_ref, run_ref):
    i = pl.program_id(0)

    @pl.when(i == 0)
    def _init():
        run_ref[...] = jnp.full((_Q, 16), _INF, jnp.float32)

    rem = _partial_distances(q_ref, s_ref, i)
    run = run_ref[...]
    ms, ccs = [], []
    cum = jnp.zeros((_Q, 1), jnp.float32)
    for _ in range(_K):
        m = jnp.minimum(jnp.min(rem, axis=1, keepdims=True),
                        jnp.min(run, axis=1, keepdims=True))
        eqc = rem == m
        eqr = run == m
        cnt = (jnp.sum(eqc.astype(jnp.float32), axis=1, keepdims=True)
               + jnp.sum(eqr.astype(jnp.float32), axis=1, keepdims=True))
        rem = jnp.where(eqc, _INF, rem)
        run = jnp.where(eqr, _INF, run)
        cum = cum + cnt
        ms.append(m)
        ccs.append(cum)
    mvals = jnp.concatenate(ms, axis=1)
    ccum = jnp.concatenate(ccs, axis=1)
    cols = [jnp.min(jnp.where(ccum > j, mvals, _INF), axis=1, keepdims=True)
            for j in range(_K)]
    cols += [jnp.full((_Q, 1), _INF, jnp.float32)] * (16 - _K)
    new_run = jnp.concatenate(cols, axis=1)
    run_ref[...] = new_run

    @pl.when(i == _NCHUNKS - 1)
    def _fini():
        d = jnp.sqrt(jnp.maximum(new_run[:, :_K] + _q2(q_ref), 0.0) + 1e-12)
        avg = jnp.sum(d, axis=1, keepdims=True) * (1.0 / _K)
        avg_ref[...] = jnp.broadcast_to(avg, (_Q, 128))
        mask = (avg > thr_ref[0, 0]).astype(jnp.float32)
        mask_ref[...] = jnp.broadcast_to(mask, (_Q, 128))


def kernel(global_embedding, geometry_latent, global_min, global_max,
           geo_embeddings, knn_threshold):
    geo_t = jnp.pad(geo_embeddings, ((0, _NPAD - _N), (0, 0))).T
    gmin = global_min.reshape(1, _GD)
    gmax = global_max.reshape(1, _GD)
    thr = jnp.asarray(knn_threshold, jnp.float32).reshape(1, 1)

    avg_b, mask_b, frac_b, flag_b = pl.pallas_call(
        _main_body,
        grid=(_NCHUNKS,),
        in_specs=[
            pl.BlockSpec((_Q, _D), lambda i: (0, 0)),
            pl.BlockSpec((_D, _CHUNK), lambda i: (0, i)),
            pl.BlockSpec((_Q, _GD), lambda i: (0, 0)),
            pl.BlockSpec((1, _GD), lambda i: (0, 0)),
            pl.BlockSpec((1, _GD), lambda i: (0, 0)),
            pl.BlockSpec((1, 1), lambda i: (0, 0)),
        ],
        out_specs=[
            pl.BlockSpec((_Q, 128), lambda i: (0, 0)),
            pl.BlockSpec((_Q, 128), lambda i: (0, 0)),
            pl.BlockSpec((1, 128), lambda i: (0, 0)),
            pl.BlockSpec((1, 128), lambda i: (0, 0)),
        ],
        out_shape=[
            jax.ShapeDtypeStruct((_Q, 128), jnp.float32),
            jax.ShapeDtypeStruct((_Q, 128), jnp.float32),
            jax.ShapeDtypeStruct((1, 128), jnp.float32),
            jax.ShapeDtypeStruct((1, 128), jnp.float32),
        ],
        scratch_shapes=[
            pltpu.VMEM((_NCHUNKS, _Q, 128), jnp.float32),
            pltpu.VMEM((_NCHUNKS, _Q, 128), jnp.float32),
            pltpu.VMEM((_NCHUNKS, _Q, 128), jnp.float32),
        ],
    )(geometry_latent, geo_t, global_embedding, gmin, gmax, thr)

    def _run_fallback(_):
        return tuple(pl.pallas_call(
            _fallback_body,
            grid=(_NCHUNKS,),
            in_specs=[
                pl.BlockSpec((_Q, _D), lambda i: (0, 0)),
                pl.BlockSpec((_D, _CHUNK), lambda i: (0, i)),
                pl.BlockSpec((1, 1), lambda i: (0, 0)),
            ],
            out_specs=[
                pl.BlockSpec((_Q, 128), lambda i: (0, 0)),
                pl.BlockSpec((_Q, 128), lambda i: (0, 0)),
            ],
            out_shape=[
                jax.ShapeDtypeStruct((_Q, 128), jnp.float32),
                jax.ShapeDtypeStruct((_Q, 128), jnp.float32),
            ],
            scratch_shapes=[pltpu.VMEM((_Q, 16), jnp.float32)],
        )(geometry_latent, geo_t, thr))

    avg_b, mask_b = jax.lax.cond(
        flag_b[0, 0] > 0.5, _run_fallback, lambda _: (avg_b, mask_b), None)

    avg = avg_b[:, 0]
    ood_mask = mask_b[:, 0].astype(bool)
    frac_oob = frac_b[0, 0]
    return (avg, ood_mask, frac_oob)
